# Initial kernel scaffold; baseline (speedup 1.0000x reference)
import jax
import jax.numpy as jnp
from jax.experimental import pallas as pl


def kernel(x, edge_index, W1, att_src1, att_dst1, b1, W2, att_src2, att_dst2, b2):
    raise NotImplementedError("write your pallas kernel here")


# speedup vs baseline: 6.8776x; 6.8776x over previous
import functools

import jax
import jax.numpy as jnp
from jax import lax
from jax.experimental import pallas as pl
from jax.experimental.pallas import tpu as pltpu
from jax.experimental.pallas import tpu_sc as plsc

N = 10000
D = 128
HID = 8
HEADS = 8
NCLS = 7

NP = 10240
RB = 128
NBLK = NP // RB

CH = 128
NSC = 2
NTIL = 16
EP = 331776
EPW = EP // (NSC * NTIL)
NCHUNK = EPW // CH
RPT = NP // NTIL
NST = RPT // 64
EB = 512
NEBLK = EP // EB

_MESH = dict(core_axis_name="c", subcore_axis_name="s")


def _lrelu(v):
    return jnp.maximum(v, 0.2 * v)


def _sc_scatter(W):
    mesh = plsc.VectorSubcoreMesh(**_MESH)

    @functools.partial(
        pl.kernel,
        mesh=mesh,
        out_type=jax.ShapeDtypeStruct((NSC, NP, W), jnp.float32),
        scratch_types=[
            pltpu.VMEM((CH,), jnp.int32),
            pltpu.VMEM((CH, W), jnp.float32),
            pltpu.VMEM((64, W), jnp.float32),
            pltpu.VMEM_SHARED((NP, W), jnp.float32),
        ],
    )
    def kern(rows_hbm, dst_hbm, out_hbm, didx, erows, zbuf, acc):
        cid = lax.axis_index("c")
        sid = lax.axis_index("s")
        rbase = sid * RPT
        z16 = jnp.zeros((16,), jnp.float32)

        def zrow(r, carry):
            for c in range(W // 16):
                zbuf[r, pl.ds(16 * c, 16)] = z16
            return carry

        lax.fori_loop(0, 64, zrow, 0)
        for t in range(NST):
            pltpu.sync_copy(zbuf, acc.at[pl.ds(rbase + t * 64, 64)])
        plsc.subcore_barrier()

        ebase = (cid * NTIL + sid) * EPW

        def chunk(i, carry):
            g = ebase + i * CH
            pltpu.sync_copy(dst_hbm.at[pl.ds(g, CH)], didx)
            pltpu.sync_copy(rows_hbm.at[pl.ds(g, CH)], erows)
            pltpu.sync_copy(erows, acc.at[didx], add=True)
            return carry

        lax.fori_loop(0, NCHUNK, chunk, 0)
        plsc.subcore_barrier()
        for t in range(NST):
            r0 = rbase + t * 64
            pltpu.sync_copy(acc.at[pl.ds(r0, 64)], zbuf)
            pltpu.sync_copy(zbuf, out_hbm.at[cid, pl.ds(r0, 64)])

    return kern


def _tc_layer1_pre(xp, W1, As1, Ad1):
    def body(x_ref, w_ref, as_ref, ad_ref, h_ref, as_o, ad_o, m_ref,
             smax, dmax):
        i = pl.program_id(0)
        h = jnp.dot(x_ref[...], w_ref[...], preferred_element_type=jnp.float32)
        asb = jnp.dot(h, as_ref[...], preferred_element_type=jnp.float32)
        adb = jnp.dot(h, ad_ref[...], preferred_element_type=jnp.float32)
        h_ref[...] = h
        as_o[...] = asb
        ad_o[...] = adb

        @pl.when(i == 0)
        def _():
            smax[...] = jnp.full((1, 16), -1e30, jnp.float32)
            dmax[...] = jnp.full((1, 16), -1e30, jnp.float32)

        smax[...] = jnp.maximum(smax[...], jnp.max(asb, axis=0, keepdims=True))
        dmax[...] = jnp.maximum(dmax[...], jnp.max(adb, axis=0, keepdims=True))

        @pl.when(i == NBLK - 1)
        def _():
            m_ref[...] = _lrelu(smax[...] + dmax[...])

    return pl.pallas_call(
        body,
        grid=(NBLK,),
        in_specs=[
            pl.BlockSpec((RB, D), lambda i: (i, 0)),
            pl.BlockSpec((D, HEADS * HID), lambda i: (0, 0)),
            pl.BlockSpec((HEADS * HID, 16), lambda i: (0, 0)),
            pl.BlockSpec((HEADS * HID, 16), lambda i: (0, 0)),
        ],
        out_specs=[
            pl.BlockSpec((RB, 64), lambda i: (i, 0)),
            pl.BlockSpec((RB, 16), lambda i: (i, 0)),
            pl.BlockSpec((RB, 16), lambda i: (i, 0)),
            pl.BlockSpec((1, 16), lambda i: (0, 0)),
        ],
        out_shape=[
            jax.ShapeDtypeStruct((NP, 64), jnp.float32),
            jax.ShapeDtypeStruct((NP, 16), jnp.float32),
            jax.ShapeDtypeStruct((NP, 16), jnp.float32),
            jax.ShapeDtypeStruct((1, 16), jnp.float32),
        ],
        scratch_shapes=[
            pltpu.VMEM((1, 16), jnp.float32),
            pltpu.VMEM((1, 16), jnp.float32),
        ],
    )(xp, W1, As1, Ad1)


def _tc_edge1(asg, adg, hg, m1, RR):
    def body(as_ref, ad_ref, h_ref, m_ref, rr_ref, o_ref):
        w = jnp.exp(_lrelu(as_ref[...] + ad_ref[...]) - m_ref[...])
        wrep = jnp.dot(w, rr_ref[...], preferred_element_type=jnp.float32)
        o_ref[...] = jnp.concatenate([h_ref[...] * wrep, w], axis=1)

    return pl.pallas_call(
        body,
        grid=(NEBLK,),
        in_specs=[
            pl.BlockSpec((EB, 16), lambda i: (i, 0)),
            pl.BlockSpec((EB, 16), lambda i: (i, 0)),
            pl.BlockSpec((EB, 64), lambda i: (i, 0)),
            pl.BlockSpec((1, 16), lambda i: (0, 0)),
            pl.BlockSpec((16, 64), lambda i: (0, 0)),
        ],
        out_specs=pl.BlockSpec((EB, 80), lambda i: (i, 0)),
        out_shape=jax.ShapeDtypeStruct((EP, 80), jnp.float32),
    )(asg, adg, hg, m1, RR)


def _tc_edge2(gg, adg, m2, P7):
    def body(g_ref, ad_ref, m_ref, p7_ref, o_ref):
        wfull = jnp.exp(_lrelu(g_ref[...] + ad_ref[...]) - m_ref[...])
        wrep = jnp.dot(wfull, p7_ref[...], preferred_element_type=jnp.float32)
        o_ref[...] = g_ref[...] * wrep

    return pl.pallas_call(
        body,
        grid=(NEBLK,),
        in_specs=[
            pl.BlockSpec((EB, 16), lambda i: (i, 0)),
            pl.BlockSpec((EB, 16), lambda i: (i, 0)),
            pl.BlockSpec((1, 16), lambda i: (0, 0)),
            pl.BlockSpec((16, 16), lambda i: (0, 0)),
        ],
        out_specs=pl.BlockSpec((EB, 16), lambda i: (i, 0)),
        out_shape=jax.ShapeDtypeStruct((EP, 16), jnp.float32),
    )(gg, adg, m2, P7)


def _tc_mid(a0, a1, b1r, W2big, Wad2, Rrep):
    def body(a0_ref, a1_ref, b1_ref, w2_ref, wad_ref, rr_ref,
             g_ref, adt_ref, mm_ref, smax, dmax):
        i = pl.program_id(0)
        acc = a0_ref[...] + a1_ref[...]
        s = acc[:, 64:72]
        s64 = jnp.dot(s, rr_ref[...], preferred_element_type=jnp.float32)
        z = acc[:, 0:64] / jnp.maximum(s64, 1e-30) + b1_ref[...]
        h1 = jnp.where(z > 0, z, jnp.exp(jnp.minimum(z, 0.0)) - 1.0)
        g = jnp.dot(h1, w2_ref[...], preferred_element_type=jnp.float32)
        lane = lax.broadcasted_iota(jnp.int32, (RB, 16), 1)
        g = g + jnp.where(lane == 8, 1.0, 0.0)
        adt = jnp.dot(h1, wad_ref[...], preferred_element_type=jnp.float32)
        g_ref[...] = g
        adt_ref[...] = adt

        rows = i * RB + lax.broadcasted_iota(jnp.int32, (RB, 16), 0)
        valid = rows < N

        @pl.when(i == 0)
        def _():
            smax[...] = jnp.full((1, 16), -1e30, jnp.float32)
            dmax[...] = jnp.full((1, 16), -1e30, jnp.float32)

        smax[...] = jnp.maximum(
            smax[...], jnp.max(jnp.where(valid, g, -1e30), axis=0, keepdims=True))
        dmax[...] = jnp.maximum(
            dmax[...], jnp.max(jnp.where(valid, adt, -1e30), axis=0, keepdims=True))

        @pl.when(i == NBLK - 1)
        def _():
            lane1 = lax.broadcasted_iota(jnp.int32, (1, 16), 1)
            a7 = jnp.max(jnp.where(lane1 == 7, smax[...], -1e30))
            d0 = jnp.max(jnp.where(lane1 == 0, dmax[...], -1e30))
            mm_ref[...] = jnp.full((1, 16), _lrelu(a7 + d0), jnp.float32)

    return pl.pallas_call(
        body,
        grid=(NBLK,),
        in_specs=[
            pl.BlockSpec((RB, 80), lambda i: (i, 0)),
            pl.BlockSpec((RB, 80), lambda i: (i, 0)),
            pl.BlockSpec((1, 64), lambda i: (0, 0)),
            pl.BlockSpec((64, 16), lambda i: (0, 0)),
            pl.BlockSpec((64, 16), lambda i: (0, 0)),
            pl.BlockSpec((8, 64), lambda i: (0, 0)),
        ],
        out_specs=[
            pl.BlockSpec((RB, 16), lambda i: (i, 0)),
            pl.BlockSpec((RB, 16), lambda i: (i, 0)),
            pl.BlockSpec((1, 16), lambda i: (0, 0)),
        ],
        out_shape=[
            jax.ShapeDtypeStruct((NP, 16), jnp.float32),
            jax.ShapeDtypeStruct((NP, 16), jnp.float32),
            jax.ShapeDtypeStruct((1, 16), jnp.float32),
        ],
        scratch_shapes=[
            pltpu.VMEM((1, 16), jnp.float32),
            pltpu.VMEM((1, 16), jnp.float32),
        ],
    )(a0, a1, b1r, W2big, Wad2, Rrep)


def _tc_post(a0, a1, b2r):
    def body(a0_ref, a1_ref, b2_ref, o_ref):
        t = a0_ref[...] + a1_ref[...]
        iota2 = lax.broadcasted_iota(jnp.int32, (16, 16), 0)
        P = jnp.where(iota2 == 8, 1.0, 0.0)
        s2 = jnp.dot(t, P, preferred_element_type=jnp.float32)
        logits = t / jnp.maximum(s2, 1e-30) + b2_ref[...]
        lane = lax.broadcasted_iota(jnp.int32, (RB, 16), 1)
        ok = lane < NCLS
        lm = jnp.max(jnp.where(ok, logits, -1e30), axis=1, keepdims=True)
        ex = jnp.where(ok, jnp.exp(logits - lm), 0.0)
        ss = jnp.sum(ex, axis=1, keepdims=True)
        o_ref[...] = logits - lm - jnp.log(ss)

    return pl.pallas_call(
        body,
        grid=(NBLK,),
        in_specs=[
            pl.BlockSpec((RB, 16), lambda i: (i, 0)),
            pl.BlockSpec((RB, 16), lambda i: (i, 0)),
            pl.BlockSpec((1, 16), lambda i: (0, 0)),
        ],
        out_specs=pl.BlockSpec((RB, 16), lambda i: (i, 0)),
        out_shape=jax.ShapeDtypeStruct((NP, 16), jnp.float32),
    )(a0, a1, b2r)


def kernel(x, edge_index, W1, att_src1, att_dst1, b1, W2, att_src2, att_dst2, b2):
    f32 = jnp.float32
    xp = jnp.pad(x, ((0, NP - N), (0, 0)))
    onehot = (jnp.arange(64)[:, None] // HID == jnp.arange(16)[None, :]).astype(f32)
    As1 = att_src1.reshape(64, 1) * onehot
    Ad1 = att_dst1.reshape(64, 1) * onehot

    loop = jnp.arange(N, dtype=edge_index.dtype)
    padi = jnp.full((EP - N - edge_index.shape[1],), N, edge_index.dtype)
    srcp = jnp.concatenate([edge_index[0], loop, padi])
    dstp = jnp.concatenate([edge_index[1], loop, padi])

    as2v = att_src2.reshape(NCLS, 1)
    ad2v = att_dst2.reshape(NCLS, 1)
    W2big = jnp.concatenate([W2, W2 @ as2v, jnp.zeros((64, 8), f32)], axis=1)
    Wad2 = jnp.concatenate(
        [jnp.tile(W2 @ ad2v, (1, 8)), jnp.zeros((64, 8), f32)], axis=1)
    Rrep = (jnp.arange(8)[:, None] == jnp.arange(64)[None, :] // HID).astype(f32)
    RR = (jnp.arange(16)[:, None] == jnp.arange(64)[None, :] // HID).astype(f32)
    P7 = (jnp.arange(16)[:, None] == 7).astype(f32) * jnp.ones((16, 16), f32)
    b1r = b1.reshape(1, 64)
    b2r = jnp.concatenate([b2, jnp.zeros((9,), f32)]).reshape(1, 16)

    H1, AS1, AD1, M1 = _tc_layer1_pre(xp, W1, As1, Ad1)
    asg = jnp.take(AS1, srcp, axis=0)
    adg = jnp.take(AD1, dstp, axis=0)
    hg = jnp.take(H1, srcp, axis=0)
    rows1 = _tc_edge1(asg, adg, hg, M1, RR)
    acc1 = _sc_scatter(80)(rows1, dstp)

    G2, ad2t, M2 = _tc_mid(acc1[0], acc1[1], b1r, W2big, Wad2, Rrep)
    gg = jnp.take(G2, srcp, axis=0)
    ad2g = jnp.take(ad2t, dstp, axis=0)
    rows2 = _tc_edge2(gg, ad2g, M2, P7)
    acc2 = _sc_scatter(16)(rows2, dstp)

    out = _tc_post(acc2[0], acc2[1], b2r)
    return out[:N, :NCLS]
